# Initial kernel scaffold; baseline (speedup 1.0000x reference)
import functools

import jax
import jax.numpy as jnp
from jax import lax
from jax.experimental import pallas as pl
from jax.experimental.pallas import tpu as pltpu
from jax.experimental.pallas import tpu_sc as plsc

_VOCAB = 1000000
_WORD_DIM = 64
_POS_DIM = 16
_BAG = 1024
_SEQ = 200
_OUT_DIM = _WORD_DIM + 2 * _POS_DIM

_NC = 2
_NS = 16
_NW = _NC * _NS

_TOKENS = _BAG * _SEQ
_PER_W = _TOKENS // _NW
_CHUNK = 128
_NCHUNK = _PER_W // _CHUNK


def _emb_kernel(widx, p1idx, p2idx, w_word, w_pos1, w_pos2, out,
                idx_v, w_buf, p1_buf, p2_buf, sem_w, sem_p1, sem_p2):
    wid = lax.axis_index("s") * _NC + lax.axis_index("c")
    pltpu.sync_copy(widx.at[wid], idx_v.at[0])
    pltpu.sync_copy(p1idx.at[wid], idx_v.at[1])
    pltpu.sync_copy(p2idx.at[wid], idx_v.at[2])

    @pl.loop(0, _NCHUNK)
    def _chunk(c):
        base = wid * _PER_W + c * _CHUNK
        cp_w = pltpu.async_copy(w_word.at[idx_v.at[0, c]], w_buf, sem_w)
        cp_p1 = pltpu.async_copy(w_pos1.at[idx_v.at[1, c]], p1_buf, sem_p1)
        cp_p2 = pltpu.async_copy(w_pos2.at[idx_v.at[2, c]], p2_buf, sem_p2)
        cp_w.wait()
        pltpu.sync_copy(w_buf, out.at[pl.ds(base, _CHUNK), pl.ds(0, _WORD_DIM)])
        cp_p1.wait()
        pltpu.sync_copy(
            p1_buf, out.at[pl.ds(base, _CHUNK), pl.ds(_WORD_DIM, _POS_DIM)])
        cp_p2.wait()
        pltpu.sync_copy(
            p2_buf,
            out.at[pl.ds(base, _CHUNK), pl.ds(_WORD_DIM + _POS_DIM, _POS_DIM)])


@jax.jit
def _run(widx, p1idx, p2idx, w_word, w_pos1, w_pos2):
    mesh = plsc.VectorSubcoreMesh(core_axis_name="c", subcore_axis_name="s")
    return pl.kernel(
        _emb_kernel,
        out_type=jax.ShapeDtypeStruct((_TOKENS, _OUT_DIM), jnp.float32),
        mesh=mesh,
        scratch_types=[
            pltpu.VMEM((3, _NCHUNK, _CHUNK), jnp.int32),
            pltpu.VMEM((_CHUNK, _WORD_DIM), jnp.float32),
            pltpu.VMEM((_CHUNK, _POS_DIM), jnp.float32),
            pltpu.VMEM((_CHUNK, _POS_DIM), jnp.float32),
            pltpu.SemaphoreType.DMA,
            pltpu.SemaphoreType.DMA,
            pltpu.SemaphoreType.DMA,
        ],
    )(widx, p1idx, p2idx, w_word, w_pos1, w_pos2)


def kernel(word, position1, position2, W_word, W_pos1, W_pos2):
    widx = word.reshape(_NW, _NCHUNK, _CHUNK)
    p1idx = position1.reshape(_NW, _NCHUNK, _CHUNK)
    p2idx = position2.reshape(_NW, _NCHUNK, _CHUNK)
    out = _run(widx, p1idx, p2idx, W_word, W_pos1, W_pos2)
    return out.reshape(_BAG, 1, _SEQ, _OUT_DIM)


# speedup vs baseline: 2.0038x; 2.0038x over previous
import functools

import jax
import jax.numpy as jnp
from jax import lax
from jax.experimental import pallas as pl
from jax.experimental.pallas import tpu as pltpu
from jax.experimental.pallas import tpu_sc as plsc

_VOCAB = 1000000
_WORD_DIM = 64
_POS_DIM = 16
_BAG = 1024
_SEQ = 200
_OUT_DIM = _WORD_DIM + 2 * _POS_DIM

_NC = 2
_NS = 16
_NW = _NC * _NS

_TOKENS = _BAG * _SEQ
_PER_W = _TOKENS // _NW
_CHUNK = 128
_NCHUNK = _PER_W // _CHUNK


def _emb_kernel(widx, p1idx, p2idx, w_word, w_pos1, w_pos2, out,
                idx_v, w_buf, p1_buf, p2_buf, sem_w, sem_p1, sem_p2):
    wid = lax.axis_index("s") * _NC + lax.axis_index("c")
    pltpu.sync_copy(widx.at[wid], idx_v.at[0])
    pltpu.sync_copy(p1idx.at[wid], idx_v.at[1])
    pltpu.sync_copy(p2idx.at[wid], idx_v.at[2])

    @pl.loop(0, _NCHUNK)
    def _chunk(c):
        base = wid * _PER_W + c * _CHUNK
        cp_w = pltpu.async_copy(w_word.at[idx_v.at[0, c]], w_buf, sem_w)
        cp_p1 = pltpu.async_copy(w_pos1.at[idx_v.at[1, c]], p1_buf, sem_p1)
        cp_p2 = pltpu.async_copy(w_pos2.at[idx_v.at[2, c]], p2_buf, sem_p2)
        cp_w.wait()
        pltpu.sync_copy(w_buf, out.at[pl.ds(base, _CHUNK), pl.ds(0, _WORD_DIM)])
        cp_p1.wait()
        pltpu.sync_copy(
            p1_buf, out.at[pl.ds(base, _CHUNK), pl.ds(_WORD_DIM, _POS_DIM)])
        cp_p2.wait()
        pltpu.sync_copy(
            p2_buf,
            out.at[pl.ds(base, _CHUNK), pl.ds(_WORD_DIM + _POS_DIM, _POS_DIM)])


@jax.jit
def _run(widx, p1idx, p2idx, w_word, w_pos1, w_pos2):
    mesh = plsc.VectorSubcoreMesh(core_axis_name="c", subcore_axis_name="s")
    return pl.kernel(
        _emb_kernel,
        out_type=jax.ShapeDtypeStruct((_TOKENS, _OUT_DIM), jnp.float32),
        mesh=mesh,
        compiler_params=pltpu.CompilerParams(use_tc_tiling_on_sc=False),
        scratch_types=[
            pltpu.VMEM((3, _NCHUNK, _CHUNK), jnp.int32),
            pltpu.VMEM((_CHUNK, _WORD_DIM), jnp.float32),
            pltpu.VMEM((_CHUNK, _POS_DIM), jnp.float32),
            pltpu.VMEM((_CHUNK, _POS_DIM), jnp.float32),
            pltpu.SemaphoreType.DMA,
            pltpu.SemaphoreType.DMA,
            pltpu.SemaphoreType.DMA,
        ],
    )(widx, p1idx, p2idx, w_word, w_pos1, w_pos2)


def kernel(word, position1, position2, W_word, W_pos1, W_pos2):
    widx = word.reshape(_NW, _NCHUNK, _CHUNK)
    p1idx = position1.reshape(_NW, _NCHUNK, _CHUNK)
    p2idx = position2.reshape(_NW, _NCHUNK, _CHUNK)
    out = _run(widx, p1idx, p2idx, W_word, W_pos1, W_pos2)
    return out.reshape(_BAG, 1, _SEQ, _OUT_DIM)


# speedup vs baseline: 2.0311x; 1.0136x over previous
import jax
import jax.numpy as jnp
from jax import lax
from jax.experimental import pallas as pl
from jax.experimental.pallas import tpu as pltpu
from jax.experimental.pallas import tpu_sc as plsc

_VOCAB = 1000000
_WORD_DIM = 64
_POS_DIM = 16
_BAG = 1024
_SEQ = 200
_OUT_DIM = _WORD_DIM + 2 * _POS_DIM

_NC = 2
_NS = 16
_NW = _NC * _NS

_TOKENS = _BAG * _SEQ
_PER_W = _TOKENS // _NW
_CHUNK = 128
_NCHUNK = _PER_W // _CHUNK
_NBUF = 2


def _emb_kernel(widx, p1idx, p2idx, w_word, w_pos1, w_pos2, out,
                idx_v, w_buf, p1_buf, p2_buf, gsem, wsem):
    wid = lax.axis_index("s") * _NC + lax.axis_index("c")
    pltpu.sync_copy(widx.at[wid], idx_v.at[0])
    pltpu.sync_copy(p1idx.at[wid], idx_v.at[1])
    pltpu.sync_copy(p2idx.at[wid], idx_v.at[2])

    def gather_descs(c, make):
        b = lax.rem(c, _NBUF)
        return [
            make(w_word.at[idx_v.at[0, c]], w_buf.at[b], gsem),
            make(w_pos1.at[idx_v.at[1, c]], p1_buf.at[b], gsem),
            make(w_pos2.at[idx_v.at[2, c]], p2_buf.at[b], gsem),
        ]

    def write_descs(c, make):
        b = lax.rem(c, _NBUF)
        rows = pl.ds(wid * _PER_W + c * _CHUNK, _CHUNK)
        return [
            make(w_buf.at[b], out.at[rows, pl.ds(0, _WORD_DIM)], wsem),
            make(p1_buf.at[b], out.at[rows, pl.ds(_WORD_DIM, _POS_DIM)],
                 wsem),
            make(p2_buf.at[b],
                 out.at[rows, pl.ds(_WORD_DIM + _POS_DIM, _POS_DIM)], wsem),
        ]

    gather_descs(0, pltpu.async_copy)

    @pl.loop(0, _NCHUNK)
    def _chunk(c):
        @pl.when(c + 1 < _NCHUNK)
        def _fire_next():
            @pl.when(c + 1 >= _NBUF)
            def _guard_buf():
                for d in write_descs(c + 1 - _NBUF, pltpu.make_async_copy):
                    d.wait()
            gather_descs(c + 1, pltpu.async_copy)

        for d in gather_descs(c, pltpu.make_async_copy):
            d.wait()
        write_descs(c, pltpu.async_copy)

    for c in range(_NCHUNK - _NBUF, _NCHUNK):
        for d in write_descs(c, pltpu.make_async_copy):
            d.wait()


@jax.jit
def _run(widx, p1idx, p2idx, w_word, w_pos1, w_pos2):
    mesh = plsc.VectorSubcoreMesh(core_axis_name="c", subcore_axis_name="s")
    return pl.kernel(
        _emb_kernel,
        out_type=jax.ShapeDtypeStruct((_TOKENS, _OUT_DIM), jnp.float32),
        mesh=mesh,
        compiler_params=pltpu.CompilerParams(use_tc_tiling_on_sc=False),
        scratch_types=[
            pltpu.VMEM((3, _NCHUNK, _CHUNK), jnp.int32),
            pltpu.VMEM((_NBUF, _CHUNK, _WORD_DIM), jnp.float32),
            pltpu.VMEM((_NBUF, _CHUNK, _POS_DIM), jnp.float32),
            pltpu.VMEM((_NBUF, _CHUNK, _POS_DIM), jnp.float32),
            pltpu.SemaphoreType.DMA,
            pltpu.SemaphoreType.DMA,
        ],
    )(widx, p1idx, p2idx, w_word, w_pos1, w_pos2)


def kernel(word, position1, position2, W_word, W_pos1, W_pos2):
    widx = word.reshape(_NW, _NCHUNK, _CHUNK)
    p1idx = position1.reshape(_NW, _NCHUNK, _CHUNK)
    p2idx = position2.reshape(_NW, _NCHUNK, _CHUNK)
    out = _run(widx, p1idx, p2idx, W_word, W_pos1, W_pos2)
    return out.reshape(_BAG, 1, _SEQ, _OUT_DIM)


# speedup vs baseline: 2.3827x; 1.1731x over previous
import jax
import jax.numpy as jnp
from jax import lax
from jax.experimental import pallas as pl
from jax.experimental.pallas import tpu as pltpu
from jax.experimental.pallas import tpu_sc as plsc

_VOCAB = 1000000
_WORD_DIM = 64
_POS_DIM = 16
_BAG = 1024
_SEQ = 200
_OUT_DIM = _WORD_DIM + 2 * _POS_DIM

_NC = 2
_NS = 16
_NW = _NC * _NS

_TOKENS = _BAG * _SEQ
_PER_W = _TOKENS // _NW
_CHUNK = 128
_NCHUNK = _PER_W // _CHUNK
_NBUF = 2

_TCOLS = 2048
_TGRID = -(-_VOCAB // _TCOLS)


def _transpose_body(wt_ref, out_ref):
    out_ref[:, 0:_WORD_DIM] = wt_ref[...].T


@jax.jit
def _relayout_word(w_t):
    return pl.pallas_call(
        _transpose_body,
        grid=(_TGRID,),
        in_specs=[pl.BlockSpec((_WORD_DIM, _TCOLS), lambda j: (0, j))],
        out_specs=pl.BlockSpec((_TCOLS, 2 * _WORD_DIM), lambda j: (j, 0)),
        out_shape=jax.ShapeDtypeStruct((_VOCAB, 2 * _WORD_DIM), jnp.float32),
    )(w_t)


def _emb_kernel(widx, p1idx, p2idx, w_word, w_pos1, w_pos2, out,
                idx_v, w_buf, p1_buf, p2_buf, gsem, wsem):
    wid = lax.axis_index("s") * _NC + lax.axis_index("c")
    pltpu.sync_copy(widx.at[wid], idx_v.at[0])
    pltpu.sync_copy(p1idx.at[wid], idx_v.at[1])
    pltpu.sync_copy(p2idx.at[wid], idx_v.at[2])

    def gather_descs(c, make):
        b = lax.rem(c, _NBUF)
        return [
            make(w_word.at[idx_v.at[0, c]], w_buf.at[b], gsem),
            make(w_pos1.at[idx_v.at[1, c]], p1_buf.at[b], gsem),
            make(w_pos2.at[idx_v.at[2, c]], p2_buf.at[b], gsem),
        ]

    def write_descs(c, make):
        b = lax.rem(c, _NBUF)
        rows = pl.ds(wid * _PER_W + c * _CHUNK, _CHUNK)
        return [
            make(w_buf.at[b], out.at[rows, pl.ds(0, _WORD_DIM)], wsem),
            make(p1_buf.at[b], out.at[rows, pl.ds(_WORD_DIM, _POS_DIM)],
                 wsem),
            make(p2_buf.at[b],
                 out.at[rows, pl.ds(_WORD_DIM + _POS_DIM, _POS_DIM)], wsem),
        ]

    gather_descs(0, pltpu.async_copy)

    @pl.loop(0, _NCHUNK)
    def _chunk(c):
        @pl.when(c + 1 < _NCHUNK)
        def _fire_next():
            @pl.when(c + 1 >= _NBUF)
            def _guard_buf():
                for d in write_descs(c + 1 - _NBUF, pltpu.make_async_copy):
                    d.wait()
            gather_descs(c + 1, pltpu.async_copy)

        for d in gather_descs(c, pltpu.make_async_copy):
            d.wait()
        write_descs(c, pltpu.async_copy)

    for c in range(_NCHUNK - _NBUF, _NCHUNK):
        for d in write_descs(c, pltpu.make_async_copy):
            d.wait()


@jax.jit
def _run(widx, p1idx, p2idx, w_word, w_pos1, w_pos2):
    mesh = plsc.VectorSubcoreMesh(core_axis_name="c", subcore_axis_name="s")
    return pl.kernel(
        _emb_kernel,
        out_type=jax.ShapeDtypeStruct((_TOKENS, _OUT_DIM), jnp.float32),
        mesh=mesh,
        compiler_params=pltpu.CompilerParams(use_tc_tiling_on_sc=False),
        scratch_types=[
            pltpu.VMEM((3, _NCHUNK, _CHUNK), jnp.int32),
            pltpu.VMEM((_NBUF, _CHUNK, _WORD_DIM), jnp.float32),
            pltpu.VMEM((_NBUF, _CHUNK, _POS_DIM), jnp.float32),
            pltpu.VMEM((_NBUF, _CHUNK, _POS_DIM), jnp.float32),
            pltpu.SemaphoreType.DMA,
            pltpu.SemaphoreType.DMA,
        ],
    )(widx, p1idx, p2idx, w_word, w_pos1, w_pos2)


def kernel(word, position1, position2, W_word, W_pos1, W_pos2):
    widx = word.reshape(_NW, _NCHUNK, _CHUNK) * 2
    p1idx = position1.reshape(_NW, _NCHUNK, _CHUNK)
    p2idx = position2.reshape(_NW, _NCHUNK, _CHUNK)
    w_pad = _relayout_word(W_word.T)
    w_lin = w_pad.reshape(-1).reshape(2 * _VOCAB, _WORD_DIM)
    out = _run(widx, p1idx, p2idx, w_lin, W_pos1, W_pos2)
    return out.reshape(_BAG, 1, _SEQ, _OUT_DIM)
